# Initial kernel scaffold; baseline (speedup 1.0000x reference)
import jax
import jax.numpy as jnp
from jax.experimental import pallas as pl


def kernel(x_cat, x_num, W_num, b_num, tables):
    raise NotImplementedError("write your pallas kernel here")


# speedup vs baseline: 8.9993x; 8.9993x over previous
import jax
import jax.numpy as jnp
from jax import lax
from jax.experimental import pallas as pl
from jax.experimental.pallas import tpu as pltpu
from jax.experimental.pallas import tpu_sc as plsc

NC = 2
NS = 16
NW = NC * NS
L = 16

B = 4096
F_NUM = 13
F_CAT = 26
CARD1 = 1001
D = 128
T_TOK = F_NUM + F_CAT

B_PER_W = B // NW
NBUF = 6
LOOKAHEAD = 4


def _tokenizer_body(tab_hbm, xcat_hbm, xnum_hbm, w_hbm, b_hbm, out_hbm,
                    idx_v, oidx_v, xnum_v, w_v, b_v, bufs, gsem, ssem):
    wid = lax.axis_index("s") * NC + lax.axis_index("c")
    b0 = wid * B_PER_W

    pltpu.sync_copy(xcat_hbm.at[:, pl.ds(b0, B_PER_W)], idx_v)
    pltpu.sync_copy(xnum_hbm.at[:, pl.ds(b0, B_PER_W)], xnum_v)
    pltpu.sync_copy(w_hbm, w_v)
    pltpu.sync_copy(b_hbm, b_v)

    iota = lax.iota(jnp.int32, L)

    def _gidx_body(f, carry):
        off = f * CARD1 + 1
        for v in range(B_PER_W // L):
            sl = pl.ds(v * L, L)
            idx_v[f, sl] = idx_v[f, sl] + off
        return carry

    lax.fori_loop(0, F_CAT, _gidx_body, 0)

    obase = [(iota + (b0 + v * L)) * T_TOK for v in range(B_PER_W // L)]

    def _oidx_body(t, carry):
        for v in range(B_PER_W // L):
            oidx_v[t, pl.ds(v * L, L)] = obase[v] + t
        return carry

    lax.fori_loop(0, T_TOK, _oidx_body, 0)

    def _fire_gather(p):
        j = p % NBUF
        return pltpu.async_copy(tab_hbm.at[idx_v.at[p]], bufs[j], gsem[j])

    def _fire_scatter(p):
        j = p % NBUF
        return pltpu.async_copy(bufs[j], out_hbm.at[oidx_v.at[F_NUM + p]],
                                ssem[j])

    gh = [None] * F_CAT
    sh = [None] * F_CAT
    for p in range(min(LOOKAHEAD, F_CAT)):
        gh[p] = _fire_gather(p)
    for p in range(F_CAT):
        gh[p].wait()
        sh[p] = _fire_scatter(p)
        q = p + LOOKAHEAD
        if q < F_CAT:
            d = q - NBUF
            if d >= 0:
                sh[d].wait()
            gh[q] = _fire_gather(q)
    for p in range(F_CAT - NBUF, F_CAT):
        if p >= 0 and sh[p] is not None:
            sh[p].wait()

    nh = [None] * F_NUM
    for i in range(F_NUM):
        j = i % 2
        if i >= 2:
            nh[i - 2].wait()
        wv = [w_v[i, pl.ds(dv * L, L)] for dv in range(D // L)]
        bv = [b_v[i, pl.ds(dv * L, L)] for dv in range(D // L)]

        def _num_body(v, carry, wv=wv, bv=bv, i=i, j=j):
            off = pl.multiple_of(v * L, L)
            xv = xnum_v[i, pl.ds(off, L)]
            for l in range(L):
                xs = xv[l]
                bl = v * L + l
                for dv in range(D // L):
                    bufs[j][bl, pl.ds(dv * L, L)] = xs * wv[dv] + bv[dv]
            return carry

        lax.fori_loop(0, B_PER_W // L, _num_body, 0)
        nh[i] = pltpu.async_copy(bufs[j], out_hbm.at[oidx_v.at[i]], ssem[j])
    nh[F_NUM - 2].wait()
    nh[F_NUM - 1].wait()


def _build_sc_call():
    mesh = plsc.VectorSubcoreMesh(
        core_axis_name="c", subcore_axis_name="s",
        num_cores=NC, num_subcores=NS)
    scratch = [
        pltpu.VMEM((F_CAT, B_PER_W), jnp.int32),
        pltpu.VMEM((T_TOK, B_PER_W), jnp.int32),
        pltpu.VMEM((F_NUM, B_PER_W), jnp.float32),
        pltpu.VMEM((F_NUM, D), jnp.float32),
        pltpu.VMEM((F_NUM, D), jnp.float32),
        [pltpu.VMEM((B_PER_W, D), jnp.float32) for _ in range(NBUF)],
        [pltpu.SemaphoreType.DMA for _ in range(NBUF)],
        [pltpu.SemaphoreType.DMA for _ in range(NBUF)],
    ]
    return pl.kernel(
        _tokenizer_body,
        out_type=jax.ShapeDtypeStruct((B * T_TOK, D), jnp.float32),
        mesh=mesh,
        scratch_types=scratch,
        name="feature_tokenizer_sc",
    )


_SC_CALL = _build_sc_call()


def kernel(x_cat, x_num, W_num, b_num, tables):
    xcat_t = x_cat.astype(jnp.int32).T
    xnum_t = x_num.T
    tab = tables.reshape(F_CAT * CARD1, D)
    out = _SC_CALL(tab, xcat_t, xnum_t, W_num, b_num)
    return out.reshape(B, T_TOK, D)


# speedup vs baseline: 14.4978x; 1.6110x over previous
import jax
import jax.numpy as jnp
from jax import lax
from jax.experimental import pallas as pl
from jax.experimental.pallas import tpu as pltpu
from jax.experimental.pallas import tpu_sc as plsc

NC = 2
NS = 16
NW = NC * NS
L = 16

B = 4096
F_NUM = 13
F_CAT = 26
CARD1 = 1001
D = 128
T_TOK = F_NUM + F_CAT

B_PER_W = B // NW
NBUF = 6
LOOKAHEAD = 4


def _tokenizer_body(tab_hbm, xcat_hbm, xnum_hbm, w_hbm, b_hbm, out_hbm,
                    idx_v, xnum_v, w_v, b_v, bufs, gsem, ssem):
    wid = lax.axis_index("s") * NC + lax.axis_index("c")
    b0 = wid * B_PER_W

    pltpu.sync_copy(xcat_hbm.at[:, pl.ds(b0, B_PER_W)], idx_v)
    pltpu.sync_copy(xnum_hbm.at[:, pl.ds(b0, B_PER_W)], xnum_v)
    pltpu.sync_copy(w_hbm, w_v)
    pltpu.sync_copy(b_hbm, b_v)

    iota = lax.iota(jnp.int32, L)

    def _gidx_body(f, carry):
        off = f * CARD1 + 1
        for v in range(B_PER_W // L):
            sl = pl.ds(v * L, L)
            idx_v[f, sl] = idx_v[f, sl] + off
        return carry

    lax.fori_loop(0, F_CAT, _gidx_body, 0)

    def _fire_gather(p):
        j = p % NBUF
        return pltpu.async_copy(tab_hbm.at[idx_v.at[p]], bufs[j], gsem[j])

    def _fire_scatter(p):
        j = p % NBUF
        return pltpu.async_copy(
            bufs[j].reshape(B_PER_W, 1, D),
            out_hbm.at[pl.ds(b0, B_PER_W), pl.ds(F_NUM + p, 1)],
            ssem[j])

    gh = [None] * F_CAT
    sh = [None] * F_CAT
    for p in range(min(LOOKAHEAD, F_CAT)):
        gh[p] = _fire_gather(p)
    for p in range(F_CAT):
        gh[p].wait()
        sh[p] = _fire_scatter(p)
        q = p + LOOKAHEAD
        if q < F_CAT:
            d = q - NBUF
            if d >= 0:
                sh[d].wait()
            gh[q] = _fire_gather(q)
    for p in range(F_CAT - NBUF, F_CAT):
        if p >= 0 and sh[p] is not None:
            sh[p].wait()

    nh = [None] * F_NUM
    for i in range(F_NUM):
        j = i % 2
        if i >= 2:
            nh[i - 2].wait()
        wv = [w_v[i, pl.ds(dv * L, L)] for dv in range(D // L)]
        bv = [b_v[i, pl.ds(dv * L, L)] for dv in range(D // L)]

        def _num_body(v, carry, wv=wv, bv=bv, i=i, j=j):
            off = pl.multiple_of(v * L, L)
            xv = xnum_v[i, pl.ds(off, L)]
            for l in range(L):
                xs = xv[l]
                bl = v * L + l
                for dv in range(D // L):
                    bufs[j][bl, pl.ds(dv * L, L)] = xs * wv[dv] + bv[dv]
            return carry

        lax.fori_loop(0, B_PER_W // L, _num_body, 0)
        nh[i] = pltpu.async_copy(
            bufs[j].reshape(B_PER_W, 1, D),
            out_hbm.at[pl.ds(b0, B_PER_W), pl.ds(i, 1)], ssem[j])
    nh[F_NUM - 2].wait()
    nh[F_NUM - 1].wait()


def _build_sc_call():
    mesh = plsc.VectorSubcoreMesh(
        core_axis_name="c", subcore_axis_name="s",
        num_cores=NC, num_subcores=NS)
    scratch = [
        pltpu.VMEM((F_CAT, B_PER_W), jnp.int32),
        pltpu.VMEM((F_NUM, B_PER_W), jnp.float32),
        pltpu.VMEM((F_NUM, D), jnp.float32),
        pltpu.VMEM((F_NUM, D), jnp.float32),
        [pltpu.VMEM((B_PER_W, D), jnp.float32) for _ in range(NBUF)],
        [pltpu.SemaphoreType.DMA for _ in range(NBUF)],
        [pltpu.SemaphoreType.DMA for _ in range(NBUF)],
    ]
    return pl.kernel(
        _tokenizer_body,
        out_type=jax.ShapeDtypeStruct((B, T_TOK, D), jnp.float32),
        mesh=mesh,
        scratch_types=scratch,
        name="feature_tokenizer_sc",
    )


_SC_CALL = _build_sc_call()


def kernel(x_cat, x_num, W_num, b_num, tables):
    xcat_t = x_cat.astype(jnp.int32).T
    xnum_t = x_num.T
    tab = tables.reshape(F_CAT * CARD1, D)
    return _SC_CALL(tab, xcat_t, xnum_t, W_num, b_num)
